# baseline (speedup 1.0000x reference)
import functools

import jax
import jax.numpy as jnp
from jax import lax
from jax.experimental import pallas as pl
from jax.experimental.pallas import tpu as pltpu
from jax.experimental.pallas import tpu_sc as plsc
from jax._src.pallas import mpmd as _mpmd

M = 100000
D = 128
B = 16384
NW = 32
ROWS = 3128
ROWS_LAST = M - (NW - 1) * ROWS
S = B // NW
L = 16


def _prep_body(b_img, b_label, b_rt, b_lr,
               o_img, o_label, o_rt, o_lr,
               stg_label, stg_rt, stg_lr, sem_cp):
    wid = lax.axis_index("c") * 16 + lax.axis_index("s")
    base = pl.multiple_of(wid * ROWS, 8)

    @pl.when(wid < NW - 1)
    def _():
        c1 = pltpu.async_copy(b_img.at[pl.ds(base, ROWS)], o_img.at[pl.ds(base, ROWS)], sem_cp)
        i2 = pltpu.async_copy(b_label.at[pl.ds(base, ROWS)], stg_label, sem_cp)
        i3 = pltpu.async_copy(b_rt.at[pl.ds(base, ROWS)], stg_rt, sem_cp)
        i4 = pltpu.async_copy(b_lr.at[pl.ds(base, ROWS)], stg_lr, sem_cp)
        i2.wait()
        i3.wait()
        i4.wait()
        c2 = pltpu.async_copy(stg_label, o_label.at[pl.ds(base, ROWS)], sem_cp)
        c3 = pltpu.async_copy(stg_rt, o_rt.at[pl.ds(base, ROWS)], sem_cp)
        c4 = pltpu.async_copy(stg_lr, o_lr.at[pl.ds(base, ROWS)], sem_cp)
        c1.wait()
        c2.wait()
        c3.wait()
        c4.wait()

    @pl.when(wid == NW - 1)
    def _():
        c1 = pltpu.async_copy(b_img.at[pl.ds(base, ROWS_LAST)], o_img.at[pl.ds(base, ROWS_LAST)], sem_cp)
        i2 = pltpu.async_copy(b_label.at[pl.ds(base, ROWS_LAST)], stg_label.at[pl.ds(0, ROWS_LAST)], sem_cp)
        i3 = pltpu.async_copy(b_rt.at[pl.ds(base, ROWS_LAST)], stg_rt.at[pl.ds(0, ROWS_LAST)], sem_cp)
        i4 = pltpu.async_copy(b_lr.at[pl.ds(base, ROWS_LAST)], stg_lr.at[pl.ds(0, ROWS_LAST)], sem_cp)
        i2.wait()
        i3.wait()
        i4.wait()
        c2 = pltpu.async_copy(stg_label.at[pl.ds(0, ROWS_LAST)], o_label.at[pl.ds(base, ROWS_LAST)], sem_cp)
        c3 = pltpu.async_copy(stg_rt.at[pl.ds(0, ROWS_LAST)], o_rt.at[pl.ds(base, ROWS_LAST)], sem_cp)
        c4 = pltpu.async_copy(stg_lr.at[pl.ds(0, ROWS_LAST)], o_lr.at[pl.ds(base, ROWS_LAST)], sem_cp)
        c1.wait()
        c2.wait()
        c3.wait()
        c4.wait()


def _apply_body(i_img, i_label, i_rt, i_lr, pmap_hbm, keys_hbm, vals_hbm, x_hbm, y_hbm,
                o_img, o_label, o_rt, o_lr,
                kv, wv, vstar, yv, zv, rows, sem_a, sem_b):
    del i_img, i_label, i_rt, i_lr
    wid = lax.axis_index("c") * 16 + lax.axis_index("s")
    ub = pl.multiple_of(wid * S, 8)

    pltpu.sync_copy(keys_hbm.at[pl.ds(ub, S)], kv)
    pltpu.async_copy(pmap_hbm.at[kv], wv, sem_a).wait()
    pltpu.async_copy(vals_hbm.at[wv], vstar, sem_a).wait()
    g1 = pltpu.async_copy(x_hbm.at[vstar], rows, sem_a)
    g2 = pltpu.async_copy(y_hbm.at[vstar], yv, sem_b)
    for t in range(S // L):
        zv[pl.ds(t * L, L)] = jnp.zeros((L,), jnp.int32)
    g1.wait()
    g2.wait()
    s1 = pltpu.async_copy(rows, o_img.at[kv], sem_a)
    s2 = pltpu.async_copy(yv, o_label.at[kv], sem_b)
    s3 = pltpu.async_copy(zv, o_rt.at[kv], sem_b)
    s4 = pltpu.async_copy(zv, o_lr.at[kv], sem_b)
    s1.wait()
    s2.wait()
    s3.wait()
    s4.wait()


@functools.cache
def _build():
    mesh = plsc.VectorSubcoreMesh(core_axis_name="c", subcore_axis_name="s")
    i32 = jnp.int32
    prep = _mpmd._mpmd_map(
        [(mesh, _prep_body)],
        (jax.ShapeDtypeStruct((M, D), jnp.float32),
         jax.ShapeDtypeStruct((M,), i32),
         jax.ShapeDtypeStruct((M,), i32),
         jax.ShapeDtypeStruct((M,), i32)),
        input_output_aliases={},
        scratch_types=[
            pltpu.VMEM((ROWS,), i32),
            pltpu.VMEM((ROWS,), i32),
            pltpu.VMEM((ROWS,), i32),
            pltpu.SemaphoreType.DMA,
        ],
    )
    apply_ = _mpmd._mpmd_map(
        [(mesh, _apply_body)],
        (jax.ShapeDtypeStruct((M, D), jnp.float32),
         jax.ShapeDtypeStruct((M,), i32),
         jax.ShapeDtypeStruct((M,), i32),
         jax.ShapeDtypeStruct((M,), i32)),
        input_output_aliases={0: 0, 1: 1, 2: 2, 3: 3},
        scratch_types=[
            pltpu.VMEM((S,), i32),
            pltpu.VMEM((S,), i32),
            pltpu.VMEM((S,), i32),
            pltpu.VMEM((S,), i32),
            pltpu.VMEM((S,), i32),
            pltpu.VMEM((S, D), jnp.float32),
            pltpu.SemaphoreType.DMA,
            pltpu.SemaphoreType.DMA,
        ],
    )
    return prep, apply_


def kernel(buffer_img, buffer_label, buffer_replay_times, buffer_last_replay,
           idx_keys, idx_vals, x, y):
    prep, apply_ = _build()
    i32 = jnp.int32
    ki = idx_keys.astype(i32)
    vi = idx_vals.astype(i32)
    iota = jnp.arange(B, dtype=i32)
    pmap = jnp.zeros((M,), i32).at[ki].max(iota)
    img0, label0, rt0, lr0 = prep(
        buffer_img,
        buffer_label.astype(i32),
        buffer_replay_times.astype(i32),
        buffer_last_replay.astype(i32),
    )
    return apply_(img0, label0, rt0, lr0, pmap, ki, vi, x, y.astype(i32))


# speedup vs baseline: 7.9424x; 7.9424x over previous
import functools

import jax
import jax.numpy as jnp
from jax import lax
from jax.experimental import pallas as pl
from jax.experimental.pallas import tpu as pltpu
from jax.experimental.pallas import tpu_sc as plsc
from jax._src.pallas import mpmd as _mpmd

M = 100000
D = 128
B = 16384
NW = 32
ROWS = 3128
ROWS_LAST = M - (NW - 1) * ROWS
S = B // NW
CHR = 128
L = 16


def _copy_rows_staged(src, dst, base, nrows, bufs, sems):
    nfull, tail = divmod(nrows, CHR)
    chunks = [(i * CHR, CHR) for i in range(nfull)]
    if tail:
        chunks.append((nfull * CHR, tail))
    pending = [None, None]
    for ci, (off, sz) in enumerate(chunks):
        b = ci % 2
        if pending[b] is not None:
            pending[b].wait()
        pltpu.async_copy(src.at[pl.ds(base + off, sz)], bufs[b].at[pl.ds(0, sz)], sems[b]).wait()
        pending[b] = pltpu.async_copy(bufs[b].at[pl.ds(0, sz)], dst.at[pl.ds(base + off, sz)], sems[2 + b])
    for p in pending:
        if p is not None:
            p.wait()


def _prep_body(b_img, b_label, b_rt, b_lr,
               o_img, o_label, o_rt, o_lr,
               stg_label, stg_rt, stg_lr, buf0, buf1, sem_cp, si0, si1, so0, so1):
    wid = lax.axis_index("c") * 16 + lax.axis_index("s")
    base = pl.multiple_of(wid * ROWS, 8)

    @pl.when(wid < NW - 1)
    def _():
        i2 = pltpu.async_copy(b_label.at[pl.ds(base, ROWS)], stg_label, sem_cp)
        i3 = pltpu.async_copy(b_rt.at[pl.ds(base, ROWS)], stg_rt, sem_cp)
        i4 = pltpu.async_copy(b_lr.at[pl.ds(base, ROWS)], stg_lr, sem_cp)
        i2.wait()
        i3.wait()
        i4.wait()
        c2 = pltpu.async_copy(stg_label, o_label.at[pl.ds(base, ROWS)], sem_cp)
        c3 = pltpu.async_copy(stg_rt, o_rt.at[pl.ds(base, ROWS)], sem_cp)
        c4 = pltpu.async_copy(stg_lr, o_lr.at[pl.ds(base, ROWS)], sem_cp)
        _copy_rows_staged(b_img, o_img, base, ROWS, (buf0, buf1), (si0, si1, so0, so1))
        c2.wait()
        c3.wait()
        c4.wait()

    @pl.when(wid == NW - 1)
    def _():
        i2 = pltpu.async_copy(b_label.at[pl.ds(base, ROWS_LAST)], stg_label.at[pl.ds(0, ROWS_LAST)], sem_cp)
        i3 = pltpu.async_copy(b_rt.at[pl.ds(base, ROWS_LAST)], stg_rt.at[pl.ds(0, ROWS_LAST)], sem_cp)
        i4 = pltpu.async_copy(b_lr.at[pl.ds(base, ROWS_LAST)], stg_lr.at[pl.ds(0, ROWS_LAST)], sem_cp)
        i2.wait()
        i3.wait()
        i4.wait()
        c2 = pltpu.async_copy(stg_label.at[pl.ds(0, ROWS_LAST)], o_label.at[pl.ds(base, ROWS_LAST)], sem_cp)
        c3 = pltpu.async_copy(stg_rt.at[pl.ds(0, ROWS_LAST)], o_rt.at[pl.ds(base, ROWS_LAST)], sem_cp)
        c4 = pltpu.async_copy(stg_lr.at[pl.ds(0, ROWS_LAST)], o_lr.at[pl.ds(base, ROWS_LAST)], sem_cp)
        _copy_rows_staged(b_img, o_img, base, ROWS_LAST, (buf0, buf1), (si0, si1, so0, so1))
        c2.wait()
        c3.wait()
        c4.wait()


def _apply_body(i_img, i_label, i_rt, i_lr, pmap_hbm, keys_hbm, vals_hbm, x_hbm, y_hbm,
                o_img, o_label, o_rt, o_lr,
                kv, wv, vstar, yv, zv, rows, sem_a, sem_b):
    del i_img, i_label, i_rt, i_lr
    wid = lax.axis_index("c") * 16 + lax.axis_index("s")
    ub = pl.multiple_of(wid * S, 8)

    pltpu.sync_copy(keys_hbm.at[pl.ds(ub, S)], kv)
    pltpu.async_copy(pmap_hbm.at[kv], wv, sem_a).wait()
    pltpu.async_copy(vals_hbm.at[wv], vstar, sem_a).wait()
    g1 = pltpu.async_copy(x_hbm.at[vstar], rows, sem_a)
    g2 = pltpu.async_copy(y_hbm.at[vstar], yv, sem_b)
    for t in range(S // L):
        zv[pl.ds(t * L, L)] = jnp.zeros((L,), jnp.int32)
    g1.wait()
    g2.wait()
    s1 = pltpu.async_copy(rows, o_img.at[kv], sem_a)
    s2 = pltpu.async_copy(yv, o_label.at[kv], sem_b)
    s3 = pltpu.async_copy(zv, o_rt.at[kv], sem_b)
    s4 = pltpu.async_copy(zv, o_lr.at[kv], sem_b)
    s1.wait()
    s2.wait()
    s3.wait()
    s4.wait()


@functools.cache
def _build():
    mesh = plsc.VectorSubcoreMesh(core_axis_name="c", subcore_axis_name="s")
    i32 = jnp.int32
    prep = _mpmd._mpmd_map(
        [(mesh, _prep_body)],
        (jax.ShapeDtypeStruct((M, D), jnp.float32),
         jax.ShapeDtypeStruct((M,), i32),
         jax.ShapeDtypeStruct((M,), i32),
         jax.ShapeDtypeStruct((M,), i32)),
        input_output_aliases={},
        scratch_types=[
            pltpu.VMEM((ROWS,), i32),
            pltpu.VMEM((ROWS,), i32),
            pltpu.VMEM((ROWS,), i32),
            pltpu.VMEM((CHR, D), jnp.float32),
            pltpu.VMEM((CHR, D), jnp.float32),
            pltpu.SemaphoreType.DMA,
            pltpu.SemaphoreType.DMA,
            pltpu.SemaphoreType.DMA,
            pltpu.SemaphoreType.DMA,
            pltpu.SemaphoreType.DMA,
        ],
    )
    apply_ = _mpmd._mpmd_map(
        [(mesh, _apply_body)],
        (jax.ShapeDtypeStruct((M, D), jnp.float32),
         jax.ShapeDtypeStruct((M,), i32),
         jax.ShapeDtypeStruct((M,), i32),
         jax.ShapeDtypeStruct((M,), i32)),
        input_output_aliases={0: 0, 1: 1, 2: 2, 3: 3},
        scratch_types=[
            pltpu.VMEM((S,), i32),
            pltpu.VMEM((S,), i32),
            pltpu.VMEM((S,), i32),
            pltpu.VMEM((S,), i32),
            pltpu.VMEM((S,), i32),
            pltpu.VMEM((S, D), jnp.float32),
            pltpu.SemaphoreType.DMA,
            pltpu.SemaphoreType.DMA,
        ],
    )
    return prep, apply_


def kernel(buffer_img, buffer_label, buffer_replay_times, buffer_last_replay,
           idx_keys, idx_vals, x, y):
    prep, apply_ = _build()
    i32 = jnp.int32
    ki = idx_keys.astype(i32)
    vi = idx_vals.astype(i32)
    iota = jnp.arange(B, dtype=i32)
    pmap = jnp.zeros((M,), i32).at[ki].max(iota)
    img0, label0, rt0, lr0 = prep(
        buffer_img,
        buffer_label.astype(i32),
        buffer_replay_times.astype(i32),
        buffer_last_replay.astype(i32),
    )
    return apply_(img0, label0, rt0, lr0, pmap, ki, vi, x, y.astype(i32))
